# baseline (speedup 1.0000x reference)
import functools

import jax
import jax.numpy as jnp
import numpy as np
from jax import lax
from jax.experimental import pallas as pl
from jax.experimental.pallas import tpu as pltpu
from jax.experimental.pallas import tpu_sc as plsc


def _pairs():
    j1, j2 = [], []
    for a in range(11):
        for b in range(11, 41):
            j1.append(a)
            j2.append(b)
    for a in range(11, 41):
        for b in range(a + 1, 41):
            if (a - 11) // 6 != (b - 11) // 6:
                j1.append(a)
                j2.append(b)
    return np.asarray(j1, dtype=np.int32), np.asarray(j2, dtype=np.int32)


_J1, _J2 = _pairs()
_NPAIR = _J1.shape[0]
_NGRP = 44
_PPAD = _NGRP * 16
_NPADL = _PPAD - _NPAIR

_NW = 32
_B = 16384
_RPW = _B // _NW


def _idx_tables():
    i1 = np.zeros((_PPAD,), dtype=np.int32)
    i2 = np.zeros((_PPAD,), dtype=np.int32)
    i1[:_NPAIR] = 3 * _J1
    i2[:_NPAIR] = 3 * _J2
    return i1, i2


_I1, _I2 = _idx_tables()

_INTERPRET = False


def _make():
    mesh = plsc.VectorSubcoreMesh(
        core_axis_name="c", subcore_axis_name="s",
        num_cores=2, num_subcores=16)
    return functools.partial(
        pl.kernel,
        out_type=jax.ShapeDtypeStruct((_NW, 16), jnp.float32),
        mesh=mesh,
        interpret=_INTERPRET,
        compiler_params=pltpu.CompilerParams(needs_layout_passes=False),
        scratch_types=[
            pltpu.VMEM((_RPW * 123,), jnp.float32),
            pltpu.VMEM((_PPAD,), jnp.int32),
            pltpu.VMEM((_PPAD,), jnp.int32),
            pltpu.VMEM((16,), jnp.float32),
        ],
    )


def _sc_loss_body(joints_hbm, i1_hbm, i2_hbm, out_hbm, jv, i1v, i2v, accv):
    wid = lax.axis_index("s") * 2 + lax.axis_index("c")
    base = wid * (_RPW * 123)
    pltpu.sync_copy(joints_hbm.at[pl.ds(base, _RPW * 123)], jv)
    pltpu.sync_copy(i1_hbm, i1v)
    pltpu.sync_copy(i2_hbm, i2v)

    def g_body(g, acc):
        i1x = i1v[pl.ds(g * 16, 16)]
        i2x = i2v[pl.ds(g * 16, 16)]
        i1y = i1x + 1
        i1z = i1x + 2
        i2y = i2x + 1
        i2z = i2x + 2

        def r_body(r, acc):
            row = jnp.full((16,), r * 123, dtype=jnp.int32)
            x1 = plsc.load_gather(jv, [row + i1x])
            y1 = plsc.load_gather(jv, [row + i1y])
            z1 = plsc.load_gather(jv, [row + i1z])
            x2 = plsc.load_gather(jv, [row + i2x])
            y2 = plsc.load_gather(jv, [row + i2y])
            z2 = plsc.load_gather(jv, [row + i2z])
            dx = x1 - x2
            dy = y1 - y2
            dz = z1 - z2
            sq = dx * dx + dy * dy + dz * dz
            return acc + jnp.maximum(36.0 - sq, 0.0)

        return lax.fori_loop(0, _RPW, r_body, acc)

    acc = lax.fori_loop(0, _NGRP, g_body, jnp.zeros((16,), jnp.float32))
    accv[...] = acc
    pltpu.sync_copy(accv, out_hbm.at[wid])


@jax.jit
def kernel(joints):
    sc_loss = _make()(_sc_loss_body)
    partials = sc_loss(joints.reshape(-1), jnp.asarray(_I1), jnp.asarray(_I2))
    pad = np.float32(_B * _NPADL * 36.0)
    return jnp.sum(partials) - pad


# speedup vs baseline: 1.5861x; 1.5861x over previous
import functools

import jax
import jax.numpy as jnp
import numpy as np
from jax import lax
from jax.experimental import pallas as pl
from jax.experimental.pallas import tpu as pltpu
from jax.experimental.pallas import tpu_sc as plsc

_NW = 32
_B = 16384
_RPW = _B // _NW
_LG = 16
_NBG = _RPW // _LG


def _tree_sum(vs):
    while len(vs) > 1:
        nxt = [vs[i] + vs[i + 1] for i in range(0, len(vs) - 1, 2)]
        if len(vs) % 2:
            nxt.append(vs[-1])
        vs = nxt
    return vs[0]


def _make():
    mesh = plsc.VectorSubcoreMesh(
        core_axis_name="c", subcore_axis_name="s",
        num_cores=2, num_subcores=16)
    return functools.partial(
        pl.kernel,
        out_type=jax.ShapeDtypeStruct((_NW, _LG), jnp.float32),
        mesh=mesh,
        compiler_params=pltpu.CompilerParams(needs_layout_passes=False),
        scratch_types=[
            pltpu.VMEM((_RPW * 123,), jnp.float32),
            pltpu.VMEM((_LG,), jnp.float32),
        ],
    )


def _sc_loss_body(joints_hbm, out_hbm, jv, accv):
    wid = lax.axis_index("s") * 2 + lax.axis_index("c")
    pltpu.sync_copy(joints_hbm.at[pl.ds(wid * (_RPW * 123), _RPW * 123)], jv)

    iota123 = lax.iota(jnp.int32, _LG) * 123

    def bg_body(bg, acc):
        base = iota123 + bg * (_LG * 123)

        def ldc(off):
            return plsc.load_gather(jv, [base + off])

        def pair_losses(held, x2, y2, z2):
            rs = []
            for (ax, ay, az) in held:
                dx = ax - x2
                dy = ay - y2
                dz = az - z2
                sq = dx * dx + dy * dy + dz * dz
                rs.append(jnp.maximum(36.0 - sq, 0.0))
            return _tree_sum(rs)

        palm = [tuple(ldc(3 * a + c) for c in range(3)) for a in range(11)]

        def b_palm(b, acc):
            off = b * 3
            x2 = ldc(off)
            y2 = ldc(off + 1)
            z2 = ldc(off + 2)
            return acc + pair_losses(palm, x2, y2, z2)

        acc = lax.fori_loop(11, 41, b_palm, acc)

        for f1 in range(4):
            fb = 11 + 6 * f1
            fingr = [tuple(ldc(3 * (fb + i) + c) for c in range(3))
                     for i in range(6)]

            def b_cross(b, acc, fingr=fingr):
                off = b * 3
                x2 = ldc(off)
                y2 = ldc(off + 1)
                z2 = ldc(off + 2)
                return acc + pair_losses(fingr, x2, y2, z2)

            acc = lax.fori_loop(fb + 6, 41, b_cross, acc)
        return acc

    acc = lax.fori_loop(0, _NBG, bg_body, jnp.zeros((_LG,), jnp.float32))
    accv[...] = acc
    pltpu.sync_copy(accv, out_hbm.at[wid])


@jax.jit
def kernel(joints):
    sc_loss = _make()(_sc_loss_body)
    partials = sc_loss(joints.reshape(-1))
    return jnp.sum(partials)


# speedup vs baseline: 2.3592x; 1.4874x over previous
import functools

import jax
import jax.numpy as jnp
import numpy as np
from jax import lax
from jax.experimental import pallas as pl
from jax.experimental.pallas import tpu as pltpu
from jax.experimental.pallas import tpu_sc as plsc

_B = 16384

_NS = 5120
_NT = _B - _NS

_NW = 32
_RPW = _NS // _NW
_LG = 16
_NBG = _RPW // _LG


def _tree_sum(vs):
    while len(vs) > 1:
        nxt = [vs[i] + vs[i + 1] for i in range(0, len(vs) - 1, 2)]
        if len(vs) % 2:
            nxt.append(vs[-1])
        vs = nxt
    return vs[0]


def _make_sc():
    mesh = plsc.VectorSubcoreMesh(
        core_axis_name="c", subcore_axis_name="s",
        num_cores=2, num_subcores=16)
    return functools.partial(
        pl.kernel,
        out_type=jax.ShapeDtypeStruct((_NW, _LG), jnp.float32),
        mesh=mesh,
        compiler_params=pltpu.CompilerParams(needs_layout_passes=False),
        scratch_types=[
            pltpu.VMEM((_RPW * 123,), jnp.float32),
            pltpu.VMEM((_LG,), jnp.float32),
        ],
    )


def _sc_loss_body(joints_hbm, out_hbm, jv, accv):
    wid = lax.axis_index("s") * 2 + lax.axis_index("c")
    pltpu.sync_copy(joints_hbm.at[pl.ds(wid * (_RPW * 123), _RPW * 123)], jv)

    iota123 = lax.iota(jnp.int32, _LG) * 123

    def bg_body(bg, acc):
        base = iota123 + bg * (_LG * 123)

        def ldc(off):
            return plsc.load_gather(jv, [base + off])

        def pair_losses(held, x2, y2, z2):
            rs = []
            for (ax, ay, az) in held:
                dx = ax - x2
                dy = ay - y2
                dz = az - z2
                sq = dx * dx + dy * dy + dz * dz
                rs.append(jnp.maximum(36.0 - sq, 0.0))
            return _tree_sum(rs)

        palm = [tuple(ldc(3 * a + c) for c in range(3)) for a in range(11)]

        def b_palm(b, acc):
            off = b * 3
            x2 = ldc(off)
            y2 = ldc(off + 1)
            z2 = ldc(off + 2)
            return acc + pair_losses(palm, x2, y2, z2)

        acc = lax.fori_loop(11, 41, b_palm, acc)

        for f1 in range(4):
            fb = 11 + 6 * f1
            fingr = [tuple(ldc(3 * (fb + i) + c) for c in range(3))
                     for i in range(6)]

            def b_cross(b, acc, fingr=fingr):
                off = b * 3
                x2 = ldc(off)
                y2 = ldc(off + 1)
                z2 = ldc(off + 2)
                return acc + pair_losses(fingr, x2, y2, z2)

            acc = lax.fori_loop(fb + 6, 41, b_cross, acc)
        return acc

    acc = lax.fori_loop(0, _NBG, bg_body, jnp.zeros((_LG,), jnp.float32))
    accv[...] = acc
    pltpu.sync_copy(accv, out_hbm.at[wid])


def _pairs():
    j1, j2 = [], []
    for a in range(11):
        for b in range(11, 41):
            j1.append(a)
            j2.append(b)
    for a in range(11, 41):
        for b in range(a + 1, 41):
            if (a - 11) // 6 != (b - 11) // 6:
                j1.append(a)
                j2.append(b)
    return np.asarray(j1, dtype=np.int32), np.asarray(j2, dtype=np.int32)


_J1, _J2 = _pairs()
_NPAIR = _J1.shape[0]
_PPAD = 704


def _diff_matrix():
    d = np.zeros((123, 3 * _PPAD), dtype=np.float32)
    for p in range(_NPAIR):
        for c in range(3):
            d[3 * _J1[p] + c, c * _PPAD + p] += 1.0
            d[3 * _J2[p] + c, c * _PPAD + p] -= 1.0
    return d


_D = _diff_matrix()


def _tc_body(x_ref, d_ref, o_ref):
    i = pl.program_id(0)

    @pl.when(i == 0)
    def _():
        o_ref[0, 0] = 0.0

    diff = jnp.dot(x_ref[:], d_ref[:], preferred_element_type=jnp.float32)
    dx = diff[:, :_PPAD]
    dy = diff[:, _PPAD:2 * _PPAD]
    dz = diff[:, 2 * _PPAD:]
    sq = dx * dx + dy * dy + dz * dz
    lane = lax.broadcasted_iota(jnp.int32, sq.shape, 1)
    loss = jnp.where(lane < _NPAIR, jnp.maximum(36.0 - sq, 0.0), 0.0)
    o_ref[0, 0] += jnp.sum(loss)


def _tc_loss(x):
    bb = 512
    grid = x.shape[0] // bb
    d = jnp.asarray(_D)
    out = pl.pallas_call(
        _tc_body,
        grid=(grid,),
        in_specs=[
            pl.BlockSpec((bb, 123), lambda i: (i, 0)),
            pl.BlockSpec((123, 3 * _PPAD), lambda i: (0, 0)),
        ],
        out_specs=pl.BlockSpec(memory_space=pltpu.SMEM),
        out_shape=jax.ShapeDtypeStruct((1, 1), jnp.float32),
    )(x, d)
    return out[0, 0]


@jax.jit
def kernel(joints):
    sc_loss = _make_sc()(_sc_loss_body)
    sc_partials = sc_loss(joints[:_NS].reshape(-1))
    tc_total = _tc_loss(joints[_NS:])
    return jnp.sum(sc_partials) + tc_total
